# baseline (speedup 1.0000x reference)
import functools

import jax
import jax.numpy as jnp
from jax.experimental import pallas as pl
from jax.experimental.pallas import tpu as pltpu

_NUM_CODES = 8192
_DIM = 256
_COMMIT = 0.25

_BM = 512
_BN = 1024


def _argmin_body(x_ref, x2_ref, w2_ref, wt_ref, out_ref, minval_ref):
    c = pl.program_id(1)
    bn = w2_ref.shape[1]
    s = jax.lax.dot_general(
        x_ref[...], wt_ref[...], (((1,), (0,)), ((), ())),
        precision=jax.lax.Precision.DEFAULT,
        preferred_element_type=jnp.float32)
    d = (x2_ref[...] + w2_ref[...]) - 2.0 * s
    bmin = jnp.min(d, axis=1, keepdims=True)
    lane = jax.lax.broadcasted_iota(jnp.int32, d.shape, 1)
    barg = jnp.min(jnp.where(d == bmin, lane, bn), axis=1, keepdims=True)
    gidx = c * bn + barg

    @pl.when(c == 0)
    def _():
        minval_ref[...] = bmin
        out_ref[...] = gidx

    @pl.when(c != 0)
    def _():
        better = bmin < minval_ref[...]
        minval_ref[...] = jnp.where(better, bmin, minval_ref[...])
        out_ref[...] = jnp.where(better, gidx, out_ref[...])


@functools.partial(jax.jit, static_argnames=("bm", "bn"))
def _vq_argmin(flat, x2, w2_row, wt, bm=_BM, bn=_BN):
    n_tok = flat.shape[0]
    grid = (n_tok // bm, _NUM_CODES // bn)
    return pl.pallas_call(
        _argmin_body,
        grid=grid,
        in_specs=[
            pl.BlockSpec((bm, _DIM), lambda t, c: (t, 0)),
            pl.BlockSpec((bm, 1), lambda t, c: (t, 0)),
            pl.BlockSpec((1, bn), lambda t, c: (0, c)),
            pl.BlockSpec((_DIM, bn), lambda t, c: (0, c)),
        ],
        out_specs=pl.BlockSpec((bm, 1), lambda t, c: (t, 0)),
        out_shape=jax.ShapeDtypeStruct((n_tok, 1), jnp.int32),
        scratch_shapes=[pltpu.VMEM((bm, 1), jnp.float32)],
        compiler_params=pltpu.CompilerParams(
            dimension_semantics=("arbitrary", "arbitrary")),
    )(flat, x2, w2_row, wt)


def _st_loss_body(a_ref, q_ref, st_ref, loss_ref):
    r = pl.program_id(0)
    diff = q_ref[...] - a_ref[...]
    st_ref[...] = a_ref[...] + diff
    part = jnp.sum(diff * diff, axis=(0, 1), keepdims=True)

    @pl.when(r == 0)
    def _():
        loss_ref[...] = part

    @pl.when(r != 0)
    def _():
        loss_ref[...] = loss_ref[...] + part


@jax.jit
def _st_loss(a2d, q2d):
    rm = 256
    grid = (a2d.shape[0] // rm,)
    return pl.pallas_call(
        _st_loss_body,
        grid=grid,
        in_specs=[
            pl.BlockSpec((rm, a2d.shape[1]), lambda r: (r, 0)),
            pl.BlockSpec((rm, a2d.shape[1]), lambda r: (r, 0)),
        ],
        out_specs=[
            pl.BlockSpec((rm, a2d.shape[1]), lambda r: (r, 0)),
            pl.BlockSpec((1, 1), lambda r: (0, 0)),
        ],
        out_shape=[
            jax.ShapeDtypeStruct(a2d.shape, jnp.float32),
            jax.ShapeDtypeStruct((1, 1), jnp.float32),
        ],
        compiler_params=pltpu.CompilerParams(
            dimension_semantics=("arbitrary",)),
    )(a2d, q2d)


def kernel(inputs, W):
    batch, chan, height, width = inputs.shape
    flat = jnp.transpose(inputs, (0, 2, 3, 1)).reshape(-1, _DIM)
    x2 = jnp.sum(flat ** 2, axis=1, keepdims=True)
    w2 = jnp.sum(W ** 2, axis=1)
    wt = W.T

    idx = _vq_argmin(flat, x2, w2.reshape(1, -1), wt)
    idx_flat = idx.reshape(-1)

    quantized = jnp.take(W, idx_flat, axis=0)
    q = jnp.transpose(quantized.reshape(batch, height, width, chan),
                      (0, 3, 1, 2))

    n_el = inputs.size
    a2d = inputs.reshape(batch * chan, height * width)
    q2d = q.reshape(batch * chan, height * width)
    st2d, loss_sum = _st_loss(a2d, q2d)
    quantized_st = st2d.reshape(inputs.shape)
    commitment_loss = _COMMIT * (loss_sum[0, 0] / n_el)
    return (quantized_st, commitment_loss, idx_flat)
